# baseline (speedup 1.0000x reference)
import jax
import jax.numpy as jnp
from jax.experimental import pallas as pl
from jax.experimental.pallas import tpu as pltpu

N = 10000
F = 128
TM = 200
STEPS = N // 2 // TM


def _body(feat_ref, w_ref, adj_t_ref, adj_b_ref, out_ref, xw_ref):
    i = pl.program_id(0)

    @pl.when(i == 0)
    def _():
        xw_ref[...] = jnp.dot(
            feat_ref[...], w_ref[...], preferred_element_type=jnp.float32
        ).astype(jnp.bfloat16)

    out_ref[0, :, :] = jnp.dot(
        adj_t_ref[...].astype(jnp.bfloat16),
        xw_ref[...],
        preferred_element_type=jnp.float32,
    )
    out_ref[1, :, :] = jnp.dot(
        adj_b_ref[...].astype(jnp.bfloat16),
        xw_ref[...],
        preferred_element_type=jnp.float32,
    )


def kernel(feat, adj, weight):
    out = pl.pallas_call(
        _body,
        grid=(STEPS,),
        in_specs=[
            pl.BlockSpec((N, F), lambda i: (0, 0)),
            pl.BlockSpec((F, F), lambda i: (0, 0)),
            pl.BlockSpec((TM, N), lambda i: (i, 0)),
            pl.BlockSpec((TM, N), lambda i: (STEPS + i, 0)),
        ],
        out_specs=pl.BlockSpec((2, TM, F), lambda i: (0, i, 0)),
        out_shape=jax.ShapeDtypeStruct((2, N // 2, F), jnp.float32),
        scratch_shapes=[pltpu.VMEM((N, F), jnp.bfloat16)],
    )(feat, weight, adj, adj)
    return out.reshape(N, F)


# speedup vs baseline: 1.0169x; 1.0169x over previous
import jax
import jax.numpy as jnp
from jax.experimental import pallas as pl
from jax.experimental.pallas import tpu as pltpu

N = 10000
F = 128
TM = 200


def _body(feat_ref, w_ref, adj_ref, out_ref, xw_ref):
    i = pl.program_id(0)

    @pl.when(i == 0)
    def _():
        xw_ref[...] = jnp.dot(
            feat_ref[...], w_ref[...], preferred_element_type=jnp.float32
        ).astype(jnp.bfloat16)

    out_ref[...] = jnp.dot(
        adj_ref[...].astype(jnp.bfloat16),
        xw_ref[...],
        preferred_element_type=jnp.float32,
    )


def kernel(feat, adj, weight):
    return pl.pallas_call(
        _body,
        grid=(N // TM,),
        in_specs=[
            pl.BlockSpec((N, F), lambda i: (0, 0)),
            pl.BlockSpec((F, F), lambda i: (0, 0)),
            pl.BlockSpec((TM, N), lambda i: (i, 0)),
        ],
        out_specs=pl.BlockSpec((TM, F), lambda i: (i, 0)),
        out_shape=jax.ShapeDtypeStruct((N, F), jnp.float32),
        scratch_shapes=[pltpu.VMEM((N, F), jnp.bfloat16)],
    )(feat, weight, adj)


# speedup vs baseline: 1.0178x; 1.0008x over previous
import jax
import jax.numpy as jnp
from jax.experimental import pallas as pl
from jax.experimental.pallas import tpu as pltpu

N = 10000
F = 128
TM = 400


def _body(feat_ref, w_ref, adj_ref, out_ref, xw_ref):
    i = pl.program_id(0)

    @pl.when(i == 0)
    def _():
        xw_ref[...] = jnp.dot(
            feat_ref[...], w_ref[...], preferred_element_type=jnp.float32
        ).astype(jnp.bfloat16)

    out_ref[...] = jnp.dot(
        adj_ref[...].astype(jnp.bfloat16),
        xw_ref[...],
        preferred_element_type=jnp.float32,
    )


def kernel(feat, adj, weight):
    return pl.pallas_call(
        _body,
        grid=(N // TM,),
        in_specs=[
            pl.BlockSpec((N, F), lambda i: (0, 0)),
            pl.BlockSpec((F, F), lambda i: (0, 0)),
            pl.BlockSpec((TM, N), lambda i: (i, 0)),
        ],
        out_specs=pl.BlockSpec((TM, F), lambda i: (i, 0)),
        out_shape=jax.ShapeDtypeStruct((N, F), jnp.float32),
        scratch_shapes=[pltpu.VMEM((N, F), jnp.bfloat16)],
    )(feat, weight, adj)
